# baseline (speedup 1.0000x reference)
import jax
import jax.numpy as jnp
from jax.experimental import pallas as pl


def _body(t_ref, s_ref, f_ref, o_ref):
    t = t_ref[...]
    s = s_ref[...]
    o_ref[0] = f_ref[0] + t[:, None, :] + s[None, :, :]


def kernel(feat, spatial_emb, temporal_emb):
    B, L, O, D = feat.shape
    return pl.pallas_call(
        _body,
        grid=(B,),
        in_specs=[
            pl.BlockSpec((L, D), lambda b: (0, 0)),
            pl.BlockSpec((O, D), lambda b: (0, 0)),
            pl.BlockSpec((1, L, O, D), lambda b: (b, 0, 0, 0)),
        ],
        out_specs=pl.BlockSpec((1, L, O, D), lambda b: (b, 0, 0, 0)),
        out_shape=jax.ShapeDtypeStruct((B, L, O, D), feat.dtype),
    )(temporal_emb, spatial_emb, feat)


# speedup vs baseline: 1.2551x; 1.2551x over previous
import jax
import jax.numpy as jnp
from jax.experimental import pallas as pl


def _pos_body(t_ref, s_ref, o_ref):
    t = t_ref[...]
    s = s_ref[...]
    o_ref[...] = t[:, None, :] + s[None, :, :]


def _add_body(p_ref, f_ref, o_ref):
    o_ref[0] = f_ref[0] + p_ref[...]


def kernel(feat, spatial_emb, temporal_emb):
    B, L, O, D = feat.shape
    OD = O * D

    pos = pl.pallas_call(
        _pos_body,
        out_shape=jax.ShapeDtypeStruct((L, O, D), feat.dtype),
    )(temporal_emb, spatial_emb)

    pos2 = pos.reshape(L, OD)
    feat2 = feat.reshape(B, L, OD)
    out = pl.pallas_call(
        _add_body,
        grid=(B,),
        in_specs=[
            pl.BlockSpec((L, OD), lambda b: (0, 0)),
            pl.BlockSpec((1, L, OD), lambda b: (b, 0, 0)),
        ],
        out_specs=pl.BlockSpec((1, L, OD), lambda b: (b, 0, 0)),
        out_shape=jax.ShapeDtypeStruct((B, L, OD), feat.dtype),
    )(pos2, feat2)
    return out.reshape(B, L, O, D)
